# initial kernel scaffold (unmeasured)
import jax
import jax.numpy as jnp
from jax import lax
from jax.experimental import pallas as pl
from jax.experimental.pallas import tpu as pltpu

N_DEV = 4
M_GLOBAL = 4096
CHUNK = M_GLOBAL // N_DEV
N_COLS = 2048


def kernel(x, w_mat):
    x = x.astype(jnp.bfloat16)
    w = w_mat.astype(jnp.bfloat16)
    partial = jnp.dot(x, w, preferred_element_type=jnp.float32)
    partial = partial.astype(jnp.bfloat16)

    def body(p_ref, out_ref, send_buf, recv_buf, send_sems, recv_sems):
        my = lax.axis_index("i")
        left = (my - 1) % N_DEV
        right = (my + 1) % N_DEV

        barrier_sem = pltpu.get_barrier_semaphore()
        for nbr in (left, right):
            pl.semaphore_signal(
                barrier_sem, inc=1,
                device_id=(nbr,), device_id_type=pl.DeviceIdType.MESH,
            )
        pl.semaphore_wait(barrier_sem, 2)

        c0 = (my - 1) % N_DEV
        send_buf[0, :, :] = p_ref[pl.ds(c0 * CHUNK, CHUNK), :]

        for h in range(N_DEV - 1):
            rdma = pltpu.make_async_remote_copy(
                src_ref=send_buf.at[h],
                dst_ref=recv_buf.at[h],
                send_sem=send_sems.at[h],
                recv_sem=recv_sems.at[h],
                device_id=(right,),
                device_id_type=pl.DeviceIdType.MESH,
            )
            rdma.start()
            rdma.wait()

            cr = (my - 2 - h) % N_DEV
            acc = (
                recv_buf[h, :, :].astype(jnp.float32)
                + p_ref[pl.ds(cr * CHUNK, CHUNK), :].astype(jnp.float32)
            )
            if h < N_DEV - 2:
                send_buf[h + 1, :, :] = acc.astype(jnp.bfloat16)
            else:
                out_ref[:, :] = jnp.maximum(acc, 0.0)

    return pl.pallas_call(
        body,
        out_shape=jax.ShapeDtypeStruct((CHUNK, N_COLS), jnp.float32),
        in_specs=[pl.BlockSpec(memory_space=pltpu.VMEM)],
        out_specs=pl.BlockSpec(memory_space=pltpu.VMEM),
        scratch_shapes=[
            pltpu.VMEM((N_DEV - 1, CHUNK, N_COLS), jnp.bfloat16),
            pltpu.VMEM((N_DEV - 1, CHUNK, N_COLS), jnp.bfloat16),
            pltpu.SemaphoreType.DMA((N_DEV - 1,)),
            pltpu.SemaphoreType.DMA((N_DEV - 1,)),
        ],
        compiler_params=pltpu.CompilerParams(collective_id=0),
    )(partial)


# baseline (device time: 187621 ns/iter reference)
import jax
import jax.numpy as jnp
from jax import lax
from jax.experimental import pallas as pl
from jax.experimental.pallas import tpu as pltpu

N_DEV = 4
M_GLOBAL = 4096
CHUNK = M_GLOBAL // N_DEV
N_COLS = 2048


def kernel(x, w_mat):
    x = x.astype(jnp.bfloat16)
    w = w_mat.astype(jnp.bfloat16)
    partial = jnp.dot(x, w, preferred_element_type=jnp.float32)
    partial = partial.astype(jnp.bfloat16)

    def body(p_ref, out_ref, send_buf, recv_buf, send_sems, recv_sems):
        my = lax.axis_index("i")
        left = (my - 1) % N_DEV
        right = (my + 1) % N_DEV

        barrier_sem = pltpu.get_barrier_semaphore()
        for nbr in (left, right):
            pl.semaphore_signal(
                barrier_sem, inc=1,
                device_id=(nbr,), device_id_type=pl.DeviceIdType.MESH,
            )
        pl.semaphore_wait(barrier_sem, 2)

        c0 = (my - 1) % N_DEV
        send_buf[0, :, :] = p_ref[pl.ds(c0 * CHUNK, CHUNK), :]

        for h in range(N_DEV - 1):
            rdma = pltpu.make_async_remote_copy(
                src_ref=send_buf.at[h],
                dst_ref=recv_buf.at[h],
                send_sem=send_sems.at[h],
                recv_sem=recv_sems.at[h],
                device_id=(right,),
                device_id_type=pl.DeviceIdType.MESH,
            )
            rdma.start()
            rdma.wait()

            cr = (my - 2 - h) % N_DEV
            acc = (
                recv_buf[h, :, :].astype(jnp.float32)
                + p_ref[pl.ds(cr * CHUNK, CHUNK), :].astype(jnp.float32)
            )
            if h < N_DEV - 2:
                send_buf[h + 1, :, :] = acc.astype(jnp.bfloat16)
            else:
                out_ref[:, :] = jnp.maximum(acc, 0.0)

    return pl.pallas_call(
        body,
        out_shape=jax.ShapeDtypeStruct((CHUNK, N_COLS), jnp.float32),
        in_specs=[pl.BlockSpec(memory_space=pltpu.VMEM)],
        out_specs=pl.BlockSpec(memory_space=pltpu.VMEM),
        scratch_shapes=[
            pltpu.VMEM((N_DEV - 1, CHUNK, N_COLS), jnp.bfloat16),
            pltpu.VMEM((N_DEV - 1, CHUNK, N_COLS), jnp.bfloat16),
            pltpu.SemaphoreType.DMA((N_DEV - 1,)),
            pltpu.SemaphoreType.DMA((N_DEV - 1,)),
        ],
        compiler_params=pltpu.CompilerParams(
            collective_id=0,
            vmem_limit_bytes=100 * 1024 * 1024,
        ),
    )(partial)


# device time: 113924 ns/iter; 1.6469x vs baseline; 1.6469x over previous
import jax
import jax.numpy as jnp
from jax import lax
from jax.experimental import pallas as pl
from jax.experimental.pallas import tpu as pltpu

N_DEV = 4
M_GLOBAL = 4096
CHUNK = M_GLOBAL // N_DEV
N_COLS = 2048
HALF = N_COLS // 2


def kernel(x, w_mat):
    x = x.astype(jnp.bfloat16)
    w = w_mat.astype(jnp.bfloat16)

    def body(x_ref, w_ref, out_ref,
             send_r, recv_r, send_l, recv_l,
             send_sems_r, recv_sems_r, send_sems_l, recv_sems_l):
        my = lax.axis_index("i")
        left = (my - 1) % N_DEV
        right = (my + 1) % N_DEV

        def mm(c, col0):
            return jnp.dot(
                x_ref[pl.ds(c * CHUNK, CHUNK), :],
                w_ref[:, pl.ds(col0, HALF)],
                preferred_element_type=jnp.float32,
            )

        barrier_sem = pltpu.get_barrier_semaphore()
        for nbr in (left, right):
            pl.semaphore_signal(
                barrier_sem, inc=1,
                device_id=(nbr,), device_id_type=pl.DeviceIdType.MESH,
            )
        pl.semaphore_wait(barrier_sem, 2)

        send_r[0, :, :] = mm((my - 1) % N_DEV, 0).astype(jnp.bfloat16)
        send_l[0, :, :] = mm((my + 1) % N_DEV, HALF).astype(jnp.bfloat16)

        prev = None
        for h in range(N_DEV - 1):
            rdma_r = pltpu.make_async_remote_copy(
                src_ref=send_r.at[h], dst_ref=recv_r.at[h],
                send_sem=send_sems_r.at[h], recv_sem=recv_sems_r.at[h],
                device_id=(right,), device_id_type=pl.DeviceIdType.MESH,
            )
            rdma_l = pltpu.make_async_remote_copy(
                src_ref=send_l.at[h], dst_ref=recv_l.at[h],
                send_sem=send_sems_l.at[h], recv_sem=recv_sems_l.at[h],
                device_id=(left,), device_id_type=pl.DeviceIdType.MESH,
            )
            rdma_r.start()
            rdma_l.start()

            pa = mm((my - 2 - h) % N_DEV, 0)
            pb = mm((my + 2 + h) % N_DEV, HALF)

            rdma_r.wait()
            rdma_l.wait()

            acc_a = recv_r[h, :, :].astype(jnp.float32) + pa
            acc_b = recv_l[h, :, :].astype(jnp.float32) + pb
            if h < N_DEV - 2:
                send_r[h + 1, :, :] = acc_a.astype(jnp.bfloat16)
                send_l[h + 1, :, :] = acc_b.astype(jnp.bfloat16)
            else:
                out_ref[:, pl.ds(0, HALF)] = jnp.maximum(acc_a, 0.0)
                out_ref[:, pl.ds(HALF, HALF)] = jnp.maximum(acc_b, 0.0)

    return pl.pallas_call(
        body,
        out_shape=jax.ShapeDtypeStruct((CHUNK, N_COLS), jnp.float32),
        in_specs=[
            pl.BlockSpec(memory_space=pltpu.VMEM),
            pl.BlockSpec(memory_space=pltpu.VMEM),
        ],
        out_specs=pl.BlockSpec(memory_space=pltpu.VMEM),
        scratch_shapes=[
            pltpu.VMEM((N_DEV - 1, CHUNK, HALF), jnp.bfloat16),
            pltpu.VMEM((N_DEV - 1, CHUNK, HALF), jnp.bfloat16),
            pltpu.VMEM((N_DEV - 1, CHUNK, HALF), jnp.bfloat16),
            pltpu.VMEM((N_DEV - 1, CHUNK, HALF), jnp.bfloat16),
            pltpu.SemaphoreType.DMA((N_DEV - 1,)),
            pltpu.SemaphoreType.DMA((N_DEV - 1,)),
            pltpu.SemaphoreType.DMA((N_DEV - 1,)),
            pltpu.SemaphoreType.DMA((N_DEV - 1,)),
        ],
        compiler_params=pltpu.CompilerParams(
            collective_id=0,
            vmem_limit_bytes=100 * 1024 * 1024,
        ),
    )(x, w)


# device time: 105068 ns/iter; 1.7857x vs baseline; 1.0843x over previous
import jax
import jax.numpy as jnp
from jax import lax
from jax.experimental import pallas as pl
from jax.experimental.pallas import tpu as pltpu

N_DEV = 4
M_GLOBAL = 4096
CHUNK = M_GLOBAL // N_DEV
N_COLS = 2048
HALF = N_COLS // 2
S = 2
SUB = CHUNK // S


def kernel(x, w_mat):
    x = x.astype(jnp.bfloat16)
    w = w_mat.astype(jnp.bfloat16)

    def body(x_ref, w_ref, out_ref,
             send_r, recv_r, send_l, recv_l,
             ss_r, rs_r, ss_l, rs_l):
        my = lax.axis_index("i")
        left = (my - 1) % N_DEV
        right = (my + 1) % N_DEV

        def mm_sub(c, s, col0):
            return jnp.dot(
                x_ref[pl.ds(c * CHUNK + s * SUB, SUB), :],
                w_ref[:, pl.ds(col0, HALF)],
                preferred_element_type=jnp.float32,
            )

        def make(h, s, go_right):
            if go_right:
                return pltpu.make_async_remote_copy(
                    src_ref=send_r.at[h, pl.ds(s * SUB, SUB), :],
                    dst_ref=recv_r.at[h, pl.ds(s * SUB, SUB), :],
                    send_sem=ss_r.at[h, s], recv_sem=rs_r.at[h, s],
                    device_id=(right,), device_id_type=pl.DeviceIdType.MESH,
                )
            return pltpu.make_async_remote_copy(
                src_ref=send_l.at[h, pl.ds(s * SUB, SUB), :],
                dst_ref=recv_l.at[h, pl.ds(s * SUB, SUB), :],
                send_sem=ss_l.at[h, s], recv_sem=rs_l.at[h, s],
                device_id=(left,), device_id_type=pl.DeviceIdType.MESH,
            )

        rd_r = [[make(h, s, True) for s in range(S)] for h in range(N_DEV - 1)]
        rd_l = [[make(h, s, False) for s in range(S)] for h in range(N_DEV - 1)]

        barrier_sem = pltpu.get_barrier_semaphore()
        for nbr in (left, right):
            pl.semaphore_signal(
                barrier_sem, inc=1,
                device_id=(nbr,), device_id_type=pl.DeviceIdType.MESH,
            )
        pl.semaphore_wait(barrier_sem, 2)

        for s in range(S):
            send_r[0, pl.ds(s * SUB, SUB), :] = (
                mm_sub((my - 1) % N_DEV, s, 0).astype(jnp.bfloat16))
            rd_r[0][s].start()
            send_l[0, pl.ds(s * SUB, SUB), :] = (
                mm_sub((my + 1) % N_DEV, s, HALF).astype(jnp.bfloat16))
            rd_l[0][s].start()

        for h in range(N_DEV - 1):
            last = h == N_DEV - 2
            for s in range(S):
                pa = mm_sub((my - 2 - h) % N_DEV, s, 0)
                rd_r[h][s].wait()
                acc_a = recv_r[h, pl.ds(s * SUB, SUB), :].astype(jnp.float32) + pa
                if not last:
                    send_r[h + 1, pl.ds(s * SUB, SUB), :] = acc_a.astype(jnp.bfloat16)
                    rd_r[h + 1][s].start()
                else:
                    out_ref[pl.ds(s * SUB, SUB), pl.ds(0, HALF)] = (
                        jnp.maximum(acc_a, 0.0))

                pb = mm_sub((my + 2 + h) % N_DEV, s, HALF)
                rd_l[h][s].wait()
                acc_b = recv_l[h, pl.ds(s * SUB, SUB), :].astype(jnp.float32) + pb
                if not last:
                    send_l[h + 1, pl.ds(s * SUB, SUB), :] = acc_b.astype(jnp.bfloat16)
                    rd_l[h + 1][s].start()
                else:
                    out_ref[pl.ds(s * SUB, SUB), pl.ds(HALF, HALF)] = (
                        jnp.maximum(acc_b, 0.0))

    return pl.pallas_call(
        body,
        out_shape=jax.ShapeDtypeStruct((CHUNK, N_COLS), jnp.float32),
        in_specs=[
            pl.BlockSpec(memory_space=pltpu.VMEM),
            pl.BlockSpec(memory_space=pltpu.VMEM),
        ],
        out_specs=pl.BlockSpec(memory_space=pltpu.VMEM),
        scratch_shapes=[
            pltpu.VMEM((N_DEV - 1, CHUNK, HALF), jnp.bfloat16),
            pltpu.VMEM((N_DEV - 1, CHUNK, HALF), jnp.bfloat16),
            pltpu.VMEM((N_DEV - 1, CHUNK, HALF), jnp.bfloat16),
            pltpu.VMEM((N_DEV - 1, CHUNK, HALF), jnp.bfloat16),
            pltpu.SemaphoreType.DMA((N_DEV - 1, S)),
            pltpu.SemaphoreType.DMA((N_DEV - 1, S)),
            pltpu.SemaphoreType.DMA((N_DEV - 1, S)),
            pltpu.SemaphoreType.DMA((N_DEV - 1, S)),
        ],
        compiler_params=pltpu.CompilerParams(
            collective_id=0,
            vmem_limit_bytes=100 * 1024 * 1024,
        ),
    )(x, w)


# device time: 97584 ns/iter; 1.9227x vs baseline; 1.0767x over previous
import jax
import jax.numpy as jnp
from jax import lax
from jax.experimental import pallas as pl
from jax.experimental.pallas import tpu as pltpu

N_DEV = 4
M_GLOBAL = 4096
CHUNK = M_GLOBAL // N_DEV
N_COLS = 2048
HALF = N_COLS // 2
S = 2
SUB = CHUNK // S


def kernel(x, w_mat):
    w_mat = w_mat.astype(jnp.bfloat16)

    def body(x_ref, w_ref, out_ref,
             send_r, recv_r, send_l, recv_l,
             ss_r, rs_r, ss_l, rs_l):
        my = lax.axis_index("i")
        left = (my - 1) % N_DEV
        right = (my + 1) % N_DEV

        def mm_sub(c, s, col0):
            return jnp.dot(
                x_ref[pl.ds(c * CHUNK + s * SUB, SUB), :].astype(jnp.bfloat16),
                w_ref[:, pl.ds(col0, HALF)],
                preferred_element_type=jnp.float32,
            )

        def make(h, s, go_right):
            if go_right:
                return pltpu.make_async_remote_copy(
                    src_ref=send_r.at[h, pl.ds(s * SUB, SUB), :],
                    dst_ref=recv_r.at[h, pl.ds(s * SUB, SUB), :],
                    send_sem=ss_r.at[h, s], recv_sem=rs_r.at[h, s],
                    device_id=(right,), device_id_type=pl.DeviceIdType.MESH,
                )
            return pltpu.make_async_remote_copy(
                src_ref=send_l.at[h, pl.ds(s * SUB, SUB), :],
                dst_ref=recv_l.at[h, pl.ds(s * SUB, SUB), :],
                send_sem=ss_l.at[h, s], recv_sem=rs_l.at[h, s],
                device_id=(left,), device_id_type=pl.DeviceIdType.MESH,
            )

        rd_r = [[make(h, s, True) for s in range(S)] for h in range(N_DEV - 1)]
        rd_l = [[make(h, s, False) for s in range(S)] for h in range(N_DEV - 1)]

        barrier_sem = pltpu.get_barrier_semaphore()
        for nbr in (left, right):
            pl.semaphore_signal(
                barrier_sem, inc=1,
                device_id=(nbr,), device_id_type=pl.DeviceIdType.MESH,
            )
        pl.semaphore_wait(barrier_sem, 2)

        for s in range(S):
            send_r[0, pl.ds(s * SUB, SUB), :] = (
                mm_sub((my - 1) % N_DEV, s, 0).astype(jnp.bfloat16))
            rd_r[0][s].start()
            send_l[0, pl.ds(s * SUB, SUB), :] = (
                mm_sub((my + 1) % N_DEV, s, HALF).astype(jnp.bfloat16))
            rd_l[0][s].start()

        for h in range(N_DEV - 1):
            last = h == N_DEV - 2
            for s in range(S):
                pa = mm_sub((my - 2 - h) % N_DEV, s, 0)
                rd_r[h][s].wait()
                acc_a = recv_r[h, pl.ds(s * SUB, SUB), :].astype(jnp.float32) + pa
                if not last:
                    send_r[h + 1, pl.ds(s * SUB, SUB), :] = acc_a.astype(jnp.bfloat16)
                    rd_r[h + 1][s].start()
                else:
                    out_ref[pl.ds(s * SUB, SUB), pl.ds(0, HALF)] = (
                        jnp.maximum(acc_a, 0.0))

                pb = mm_sub((my + 2 + h) % N_DEV, s, HALF)
                rd_l[h][s].wait()
                acc_b = recv_l[h, pl.ds(s * SUB, SUB), :].astype(jnp.float32) + pb
                if not last:
                    send_l[h + 1, pl.ds(s * SUB, SUB), :] = acc_b.astype(jnp.bfloat16)
                    rd_l[h + 1][s].start()
                else:
                    out_ref[pl.ds(s * SUB, SUB), pl.ds(HALF, HALF)] = (
                        jnp.maximum(acc_b, 0.0))

    return pl.pallas_call(
        body,
        out_shape=jax.ShapeDtypeStruct((CHUNK, N_COLS), jnp.float32),
        in_specs=[
            pl.BlockSpec(memory_space=pltpu.VMEM),
            pl.BlockSpec(memory_space=pltpu.VMEM),
        ],
        out_specs=pl.BlockSpec(memory_space=pltpu.VMEM),
        scratch_shapes=[
            pltpu.VMEM((N_DEV - 1, CHUNK, HALF), jnp.bfloat16),
            pltpu.VMEM((N_DEV - 1, CHUNK, HALF), jnp.bfloat16),
            pltpu.VMEM((N_DEV - 1, CHUNK, HALF), jnp.bfloat16),
            pltpu.VMEM((N_DEV - 1, CHUNK, HALF), jnp.bfloat16),
            pltpu.SemaphoreType.DMA((N_DEV - 1, S)),
            pltpu.SemaphoreType.DMA((N_DEV - 1, S)),
            pltpu.SemaphoreType.DMA((N_DEV - 1, S)),
            pltpu.SemaphoreType.DMA((N_DEV - 1, S)),
        ],
        compiler_params=pltpu.CompilerParams(
            collective_id=0,
            vmem_limit_bytes=100 * 1024 * 1024,
        ),
    )(x, w_mat)


# device time: 96189 ns/iter; 1.9505x vs baseline; 1.0145x over previous
import jax
import jax.numpy as jnp
from jax import lax
from jax.experimental import pallas as pl
from jax.experimental.pallas import tpu as pltpu

N_DEV = 4
M_GLOBAL = 4096
CHUNK = M_GLOBAL // N_DEV
N_COLS = 2048
HALF = N_COLS // 2
S = 2
SUB = CHUNK // S


def kernel(x, w_mat):
    w_mat = w_mat.astype(jnp.bfloat16)

    def body(x_ref, w_ref, out_ref,
             send_r, recv_r, send_l, recv_l, out_stage,
             ss_r, rs_r, ss_l, rs_l, copy_sems):
        my = lax.axis_index("i")
        left = (my - 1) % N_DEV
        right = (my + 1) % N_DEV

        def mm_sub(c, s, col0):
            return jnp.dot(
                x_ref[pl.ds(c * CHUNK + s * SUB, SUB), :].astype(jnp.bfloat16),
                w_ref[:, pl.ds(col0, HALF)],
                preferred_element_type=jnp.float32,
            )

        def make(h, s, go_right):
            if go_right:
                return pltpu.make_async_remote_copy(
                    src_ref=send_r.at[h, pl.ds(s * SUB, SUB), :],
                    dst_ref=recv_r.at[h, pl.ds(s * SUB, SUB), :],
                    send_sem=ss_r.at[h, s], recv_sem=rs_r.at[h, s],
                    device_id=(right,), device_id_type=pl.DeviceIdType.MESH,
                )
            return pltpu.make_async_remote_copy(
                src_ref=send_l.at[h, pl.ds(s * SUB, SUB), :],
                dst_ref=recv_l.at[h, pl.ds(s * SUB, SUB), :],
                send_sem=ss_l.at[h, s], recv_sem=rs_l.at[h, s],
                device_id=(left,), device_id_type=pl.DeviceIdType.MESH,
            )

        rd_r = [[make(h, s, True) for s in range(S)] for h in range(N_DEV - 1)]
        rd_l = [[make(h, s, False) for s in range(S)] for h in range(N_DEV - 1)]

        barrier_sem = pltpu.get_barrier_semaphore()
        for nbr in (left, right):
            pl.semaphore_signal(
                barrier_sem, inc=1,
                device_id=(nbr,), device_id_type=pl.DeviceIdType.MESH,
            )
        pl.semaphore_wait(barrier_sem, 2)

        for s in range(S):
            send_r[0, pl.ds(s * SUB, SUB), :] = (
                mm_sub((my - 1) % N_DEV, s, 0).astype(jnp.bfloat16))
            rd_r[0][s].start()
            send_l[0, pl.ds(s * SUB, SUB), :] = (
                mm_sub((my + 1) % N_DEV, s, HALF).astype(jnp.bfloat16))
            rd_l[0][s].start()

        out_copies = [None, None]

        def emit_out(slot, s, col0, val):
            if out_copies[slot] is not None:
                out_copies[slot].wait()
            out_stage[slot, :, :] = val
            cp = pltpu.make_async_copy(
                out_stage.at[slot],
                out_ref.at[pl.ds(s * SUB, SUB), pl.ds(col0, HALF)],
                copy_sems.at[slot],
            )
            cp.start()
            out_copies[slot] = cp

        for h in range(N_DEV - 1):
            last = h == N_DEV - 2
            for s in range(S):
                pa = mm_sub((my - 2 - h) % N_DEV, s, 0)
                rd_r[h][s].wait()
                acc_a = recv_r[h, pl.ds(s * SUB, SUB), :].astype(jnp.float32) + pa
                if not last:
                    send_r[h + 1, pl.ds(s * SUB, SUB), :] = acc_a.astype(jnp.bfloat16)
                    rd_r[h + 1][s].start()
                else:
                    emit_out(0, s, 0, jnp.maximum(acc_a, 0.0))

                pb = mm_sub((my + 2 + h) % N_DEV, s, HALF)
                rd_l[h][s].wait()
                acc_b = recv_l[h, pl.ds(s * SUB, SUB), :].astype(jnp.float32) + pb
                if not last:
                    send_l[h + 1, pl.ds(s * SUB, SUB), :] = acc_b.astype(jnp.bfloat16)
                    rd_l[h + 1][s].start()
                else:
                    emit_out(1, s, HALF, jnp.maximum(acc_b, 0.0))

        for cp in out_copies:
            cp.wait()

    return pl.pallas_call(
        body,
        out_shape=jax.ShapeDtypeStruct((CHUNK, N_COLS), jnp.float32),
        in_specs=[
            pl.BlockSpec(memory_space=pltpu.VMEM),
            pl.BlockSpec(memory_space=pltpu.VMEM),
        ],
        out_specs=pl.BlockSpec(memory_space=pl.ANY),
        scratch_shapes=[
            pltpu.VMEM((N_DEV - 1, CHUNK, HALF), jnp.bfloat16),
            pltpu.VMEM((N_DEV - 1, CHUNK, HALF), jnp.bfloat16),
            pltpu.VMEM((N_DEV - 1, CHUNK, HALF), jnp.bfloat16),
            pltpu.VMEM((N_DEV - 1, CHUNK, HALF), jnp.bfloat16),
            pltpu.VMEM((2, SUB, HALF), jnp.float32),
            pltpu.SemaphoreType.DMA((N_DEV - 1, S)),
            pltpu.SemaphoreType.DMA((N_DEV - 1, S)),
            pltpu.SemaphoreType.DMA((N_DEV - 1, S)),
            pltpu.SemaphoreType.DMA((N_DEV - 1, S)),
            pltpu.SemaphoreType.DMA((2,)),
        ],
        compiler_params=pltpu.CompilerParams(
            collective_id=0,
            vmem_limit_bytes=100 * 1024 * 1024,
        ),
    )(x, w_mat)


# device time: 87936 ns/iter; 2.1336x vs baseline; 1.0939x over previous
import jax
import jax.numpy as jnp
from jax import lax
from jax.experimental import pallas as pl
from jax.experimental.pallas import tpu as pltpu

N_DEV = 4
M_GLOBAL = 4096
CHUNK = M_GLOBAL // N_DEV
N_COLS = 2048
HALF = N_COLS // 2
S = 2
SUB = CHUNK // S


def kernel(x, w_mat):
    def body(x_hbm, w_hbm, out_ref,
             x_vmem, w_vmem, w_bf,
             send_r, recv_r, send_l, recv_l, out_stage,
             x_sems, w_sem,
             ss_r, rs_r, ss_l, rs_l, copy_sems):
        my = lax.axis_index("i")
        left = (my - 1) % N_DEV
        right = (my + 1) % N_DEV

        def mm_sub(c, s, col0):
            return jnp.dot(
                x_vmem[pl.ds(c * CHUNK + s * SUB, SUB), :].astype(jnp.bfloat16),
                w_bf[:, pl.ds(col0, HALF)],
                preferred_element_type=jnp.float32,
            )

        def make(h, s, go_right):
            if go_right:
                return pltpu.make_async_remote_copy(
                    src_ref=send_r.at[h, pl.ds(s * SUB, SUB), :],
                    dst_ref=recv_r.at[h, pl.ds(s * SUB, SUB), :],
                    send_sem=ss_r.at[h, s], recv_sem=rs_r.at[h, s],
                    device_id=(right,), device_id_type=pl.DeviceIdType.MESH,
                )
            return pltpu.make_async_remote_copy(
                src_ref=send_l.at[h, pl.ds(s * SUB, SUB), :],
                dst_ref=recv_l.at[h, pl.ds(s * SUB, SUB), :],
                send_sem=ss_l.at[h, s], recv_sem=rs_l.at[h, s],
                device_id=(left,), device_id_type=pl.DeviceIdType.MESH,
            )

        rd_r = [[make(h, s, True) for s in range(S)] for h in range(N_DEV - 1)]
        rd_l = [[make(h, s, False) for s in range(S)] for h in range(N_DEV - 1)]

        w_cp = pltpu.make_async_copy(w_hbm, w_vmem, w_sem)
        w_cp.start()
        x_cps = []
        for k, c in enumerate(((my - 1) % N_DEV, (my + 1) % N_DEV,
                               (my + 2) % N_DEV, my)):
            cp = pltpu.make_async_copy(
                x_hbm.at[pl.ds(c * CHUNK, CHUNK), :],
                x_vmem.at[pl.ds(c * CHUNK, CHUNK), :],
                x_sems.at[k],
            )
            cp.start()
            x_cps.append(cp)

        barrier_sem = pltpu.get_barrier_semaphore()
        for nbr in (left, right):
            pl.semaphore_signal(
                barrier_sem, inc=1,
                device_id=(nbr,), device_id_type=pl.DeviceIdType.MESH,
            )
        pl.semaphore_wait(barrier_sem, 2)

        w_cp.wait()
        w_bf[:, pl.ds(0, HALF)] = w_vmem[:, pl.ds(0, HALF)].astype(jnp.bfloat16)
        x_cps[0].wait()

        send_r[0, pl.ds(0, SUB), :] = (
            mm_sub((my - 1) % N_DEV, 0, 0).astype(jnp.bfloat16))
        rd_r[0][0].start()
        w_bf[:, pl.ds(HALF, HALF)] = (
            w_vmem[:, pl.ds(HALF, HALF)].astype(jnp.bfloat16))
        x_cps[1].wait()
        send_l[0, pl.ds(0, SUB), :] = (
            mm_sub((my + 1) % N_DEV, 0, HALF).astype(jnp.bfloat16))
        rd_l[0][0].start()
        send_r[0, pl.ds(SUB, SUB), :] = (
            mm_sub((my - 1) % N_DEV, 1, 0).astype(jnp.bfloat16))
        rd_r[0][1].start()
        send_l[0, pl.ds(SUB, SUB), :] = (
            mm_sub((my + 1) % N_DEV, 1, HALF).astype(jnp.bfloat16))
        rd_l[0][1].start()

        x_cps[2].wait()

        out_copies = [None, None]

        def emit_out(slot, s, col0, val):
            if out_copies[slot] is not None:
                out_copies[slot].wait()
            out_stage[slot, :, :] = val
            cp = pltpu.make_async_copy(
                out_stage.at[slot],
                out_ref.at[pl.ds(s * SUB, SUB), pl.ds(col0, HALF)],
                copy_sems.at[slot],
            )
            cp.start()
            out_copies[slot] = cp

        for h in range(N_DEV - 1):
            last = h == N_DEV - 2
            if last:
                x_cps[3].wait()
            for s in range(S):
                pa = mm_sub((my - 2 - h) % N_DEV, s, 0)
                rd_r[h][s].wait()
                acc_a = recv_r[h, pl.ds(s * SUB, SUB), :].astype(jnp.float32) + pa
                if not last:
                    send_r[h + 1, pl.ds(s * SUB, SUB), :] = acc_a.astype(jnp.bfloat16)
                    rd_r[h + 1][s].start()
                else:
                    emit_out(0, s, 0, jnp.maximum(acc_a, 0.0))

                pb = mm_sub((my + 2 + h) % N_DEV, s, HALF)
                rd_l[h][s].wait()
                acc_b = recv_l[h, pl.ds(s * SUB, SUB), :].astype(jnp.float32) + pb
                if not last:
                    send_l[h + 1, pl.ds(s * SUB, SUB), :] = acc_b.astype(jnp.bfloat16)
                    rd_l[h + 1][s].start()
                else:
                    emit_out(1, s, HALF, jnp.maximum(acc_b, 0.0))

        for cp in out_copies:
            cp.wait()

    return pl.pallas_call(
        body,
        out_shape=jax.ShapeDtypeStruct((CHUNK, N_COLS), jnp.float32),
        in_specs=[
            pl.BlockSpec(memory_space=pl.ANY),
            pl.BlockSpec(memory_space=pl.ANY),
        ],
        out_specs=pl.BlockSpec(memory_space=pl.ANY),
        scratch_shapes=[
            pltpu.VMEM((M_GLOBAL, 1024), jnp.float32),
            pltpu.VMEM((1024, N_COLS), jnp.float32),
            pltpu.VMEM((1024, N_COLS), jnp.bfloat16),
            pltpu.VMEM((N_DEV - 1, CHUNK, HALF), jnp.bfloat16),
            pltpu.VMEM((N_DEV - 1, CHUNK, HALF), jnp.bfloat16),
            pltpu.VMEM((N_DEV - 1, CHUNK, HALF), jnp.bfloat16),
            pltpu.VMEM((N_DEV - 1, CHUNK, HALF), jnp.bfloat16),
            pltpu.VMEM((2, SUB, HALF), jnp.float32),
            pltpu.SemaphoreType.DMA((4,)),
            pltpu.SemaphoreType.DMA,
            pltpu.SemaphoreType.DMA((N_DEV - 1, S)),
            pltpu.SemaphoreType.DMA((N_DEV - 1, S)),
            pltpu.SemaphoreType.DMA((N_DEV - 1, S)),
            pltpu.SemaphoreType.DMA((N_DEV - 1, S)),
            pltpu.SemaphoreType.DMA((2,)),
        ],
        compiler_params=pltpu.CompilerParams(
            collective_id=0,
            vmem_limit_bytes=100 * 1024 * 1024,
        ),
    )(x, w_mat)


# device time: 86172 ns/iter; 2.1773x vs baseline; 1.0205x over previous
import jax
import jax.numpy as jnp
from jax import lax
from jax.experimental import pallas as pl
from jax.experimental.pallas import tpu as pltpu

N_DEV = 4
M_GLOBAL = 4096
CHUNK = M_GLOBAL // N_DEV
N_COLS = 2048
HALF = N_COLS // 2
S = 4
SUB = CHUNK // S


def kernel(x, w_mat):
    def body(x_hbm, w_hbm, out_ref,
             x_vmem, w_vmem, w_bf,
             send_r, recv_r, send_l, recv_l, out_stage,
             x_sems, w_sem,
             ss_r, rs_r, ss_l, rs_l, copy_sems):
        my = lax.axis_index("i")
        left = (my - 1) % N_DEV
        right = (my + 1) % N_DEV

        def mm_sub(c, s, col0):
            return jnp.dot(
                x_vmem[pl.ds(c * CHUNK + s * SUB, SUB), :].astype(jnp.bfloat16),
                w_bf[:, pl.ds(col0, HALF)],
                preferred_element_type=jnp.float32,
            )

        def make(h, s, go_right):
            if go_right:
                return pltpu.make_async_remote_copy(
                    src_ref=send_r.at[h, pl.ds(s * SUB, SUB), :],
                    dst_ref=recv_r.at[h, pl.ds(s * SUB, SUB), :],
                    send_sem=ss_r.at[h, s], recv_sem=rs_r.at[h, s],
                    device_id=(right,), device_id_type=pl.DeviceIdType.MESH,
                )
            return pltpu.make_async_remote_copy(
                src_ref=send_l.at[h, pl.ds(s * SUB, SUB), :],
                dst_ref=recv_l.at[h, pl.ds(s * SUB, SUB), :],
                send_sem=ss_l.at[h, s], recv_sem=rs_l.at[h, s],
                device_id=(left,), device_id_type=pl.DeviceIdType.MESH,
            )

        rd_r = [[make(h, s, True) for s in range(S)] for h in range(N_DEV - 1)]
        rd_l = [[make(h, s, False) for s in range(S)] for h in range(N_DEV - 1)]

        w_cp_a = pltpu.make_async_copy(
            w_hbm.at[:, pl.ds(0, HALF)], w_vmem.at[:, pl.ds(0, HALF)],
            w_sem.at[0])
        w_cp_b = pltpu.make_async_copy(
            w_hbm.at[:, pl.ds(HALF, HALF)], w_vmem.at[:, pl.ds(HALF, HALF)],
            w_sem.at[1])
        w_cp_a.start()
        w_cp_b.start()
        x_cps = []
        for k, c in enumerate(((my - 1) % N_DEV, (my + 1) % N_DEV,
                               (my + 2) % N_DEV, my)):
            cp = pltpu.make_async_copy(
                x_hbm.at[pl.ds(c * CHUNK, CHUNK), :],
                x_vmem.at[pl.ds(c * CHUNK, CHUNK), :],
                x_sems.at[k],
            )
            cp.start()
            x_cps.append(cp)

        barrier_sem = pltpu.get_barrier_semaphore()
        for nbr in (left, right):
            pl.semaphore_signal(
                barrier_sem, inc=1,
                device_id=(nbr,), device_id_type=pl.DeviceIdType.MESH,
            )
        pl.semaphore_wait(barrier_sem, 2)

        w_cp_a.wait()
        w_bf[:, pl.ds(0, HALF)] = w_vmem[:, pl.ds(0, HALF)].astype(jnp.bfloat16)
        x_cps[0].wait()

        send_r[0, pl.ds(0, SUB), :] = (
            mm_sub((my - 1) % N_DEV, 0, 0).astype(jnp.bfloat16))
        rd_r[0][0].start()
        w_cp_b.wait()
        w_bf[:, pl.ds(HALF, HALF)] = (
            w_vmem[:, pl.ds(HALF, HALF)].astype(jnp.bfloat16))
        x_cps[1].wait()
        send_l[0, pl.ds(0, SUB), :] = (
            mm_sub((my + 1) % N_DEV, 0, HALF).astype(jnp.bfloat16))
        rd_l[0][0].start()
        for s in range(1, S):
            send_r[0, pl.ds(s * SUB, SUB), :] = (
                mm_sub((my - 1) % N_DEV, s, 0).astype(jnp.bfloat16))
            rd_r[0][s].start()
            send_l[0, pl.ds(s * SUB, SUB), :] = (
                mm_sub((my + 1) % N_DEV, s, HALF).astype(jnp.bfloat16))
            rd_l[0][s].start()

        x_cps[2].wait()

        out_copies = [None, None]

        def emit_out(slot, s, col0, val):
            if out_copies[slot] is not None:
                out_copies[slot].wait()
            out_stage[slot, :, :] = val
            cp = pltpu.make_async_copy(
                out_stage.at[slot],
                out_ref.at[pl.ds(s * SUB, SUB), pl.ds(col0, HALF)],
                copy_sems.at[slot],
            )
            cp.start()
            out_copies[slot] = cp

        for h in range(N_DEV - 1):
            last = h == N_DEV - 2
            if last:
                x_cps[3].wait()
            for s in range(S):
                pa = mm_sub((my - 2 - h) % N_DEV, s, 0)
                rd_r[h][s].wait()
                acc_a = recv_r[h, pl.ds(s * SUB, SUB), :].astype(jnp.float32) + pa
                if not last:
                    send_r[h + 1, pl.ds(s * SUB, SUB), :] = acc_a.astype(jnp.bfloat16)
                    rd_r[h + 1][s].start()
                else:
                    emit_out(0, s, 0, jnp.maximum(acc_a, 0.0))

                pb = mm_sub((my + 2 + h) % N_DEV, s, HALF)
                rd_l[h][s].wait()
                acc_b = recv_l[h, pl.ds(s * SUB, SUB), :].astype(jnp.float32) + pb
                if not last:
                    send_l[h + 1, pl.ds(s * SUB, SUB), :] = acc_b.astype(jnp.bfloat16)
                    rd_l[h + 1][s].start()
                else:
                    emit_out(1, s, HALF, jnp.maximum(acc_b, 0.0))

        for cp in out_copies:
            cp.wait()

    return pl.pallas_call(
        body,
        out_shape=jax.ShapeDtypeStruct((CHUNK, N_COLS), jnp.float32),
        in_specs=[
            pl.BlockSpec(memory_space=pl.ANY),
            pl.BlockSpec(memory_space=pl.ANY),
        ],
        out_specs=pl.BlockSpec(memory_space=pl.ANY),
        scratch_shapes=[
            pltpu.VMEM((M_GLOBAL, 1024), jnp.float32),
            pltpu.VMEM((1024, N_COLS), jnp.float32),
            pltpu.VMEM((1024, N_COLS), jnp.bfloat16),
            pltpu.VMEM((N_DEV - 1, CHUNK, HALF), jnp.bfloat16),
            pltpu.VMEM((N_DEV - 1, CHUNK, HALF), jnp.bfloat16),
            pltpu.VMEM((N_DEV - 1, CHUNK, HALF), jnp.bfloat16),
            pltpu.VMEM((N_DEV - 1, CHUNK, HALF), jnp.bfloat16),
            pltpu.VMEM((2, SUB, HALF), jnp.float32),
            pltpu.SemaphoreType.DMA((4,)),
            pltpu.SemaphoreType.DMA((2,)),
            pltpu.SemaphoreType.DMA((N_DEV - 1, S)),
            pltpu.SemaphoreType.DMA((N_DEV - 1, S)),
            pltpu.SemaphoreType.DMA((N_DEV - 1, S)),
            pltpu.SemaphoreType.DMA((N_DEV - 1, S)),
            pltpu.SemaphoreType.DMA((2,)),
        ],
        compiler_params=pltpu.CompilerParams(
            collective_id=0,
            vmem_limit_bytes=100 * 1024 * 1024,
        ),
    )(x, w_mat)
